# Initial kernel scaffold; baseline (speedup 1.0000x reference)
import jax
import jax.numpy as jnp
from jax.experimental import pallas as pl


def kernel(x, edge_index, batch, W_g1, b_g1, W_g2, b_g2, W_d1, b_d1, W_d2, b_d2, W_e, b_e):
    raise NotImplementedError("write your pallas kernel here")


# speedup vs baseline: 2.6345x; 2.6345x over previous
import functools

import jax
import jax.numpy as jnp
from jax.experimental import pallas as pl
from jax.experimental.pallas import tpu as pltpu

N = 10000
BM = 512
BN = 1024


def _adj_body(l_ref, lt_ref, out_ref):
    acc = jax.lax.dot_general(
        l_ref[...], lt_ref[...], (((1,), (0,)), ((), ())),
        preferred_element_type=jnp.float32)
    out_ref[...] = jax.nn.sigmoid(acc)


def _adj_recon(el):
    elt = el.T
    n = el.shape[0]
    grid = (pl.cdiv(n, BM), pl.cdiv(n, BN))
    return pl.pallas_call(
        _adj_body,
        grid=grid,
        in_specs=[
            pl.BlockSpec((BM, el.shape[1]), lambda i, j: (i, 0)),
            pl.BlockSpec((el.shape[1], BN), lambda i, j: (0, j)),
        ],
        out_specs=pl.BlockSpec((BM, BN), lambda i, j: (i, j)),
        out_shape=jax.ShapeDtypeStruct((n, n), jnp.float32),
    )(el, elt)


def _gcn_conv(x, src, dst, dinv, W, b):
    n = x.shape[0]
    y = (x @ W) * dinv[:, None]
    agg = jax.ops.segment_sum(y[src], dst, num_segments=n)
    return (agg + y) * dinv[:, None] + b


def kernel(x, edge_index, batch, W_g1, b_g1, W_g2, b_g2, W_d1, b_d1, W_d2, b_d2, W_e, b_e):
    n = x.shape[0]
    src = edge_index[0]
    dst = edge_index[1]
    deg = jax.ops.segment_sum(jnp.ones(src.shape[0], dtype=x.dtype), dst,
                              num_segments=n) + 1.0
    dinv = deg ** -0.5

    z = jax.nn.relu(_gcn_conv(x, src, dst, dinv, W_g1, b_g1))
    z = jax.nn.relu(_gcn_conv(z, src, dst, dinv, W_g2, b_g2))

    h = jax.nn.relu(z @ W_d1 + b_d1)
    x_recon = h @ W_d2 + b_d2
    edge_logits = z @ W_e + b_e
    adj_recon = _adj_recon(edge_logits)
    z_g = jax.ops.segment_max(z, batch, num_segments=64)
    return (z, z_g, x_recon, adj_recon)


# speedup vs baseline: 7.2209x; 2.7409x over previous
import functools

import jax
import jax.numpy as jnp
from jax import lax
from jax.experimental import pallas as pl
from jax.experimental.pallas import tpu as pltpu
from jax.experimental.pallas import tpu_sc as plsc

N = 10000
E = 160000
NW = 32
NSUB = 16
EPW = E // NW
K = 128
CH = (EPW + K - 1) // K
EPW_PAD = CH * K
NPAD = 10240
ROWS_PER_SUB = NPAD // NSUB
TRASH = NPAD - 8

BM = 512
BN = 1024


def _pad_idx(idx, fill):
    r = idx.reshape(NW, EPW)
    pad = jnp.full((NW, EPW_PAD - EPW), fill, dtype=idx.dtype)
    return jnp.concatenate([r, pad], axis=1).reshape(NW, CH, K).astype(jnp.int32)


def _agg_body(y_hbm, isrc_hbm, idst_hbm, out_hbm,
              isrc_v, idst_v, rows_v, zbuf_v, acc_sh, sem_g):
    cid = lax.axis_index("c")
    sid = lax.axis_index("s")
    w = cid * NSUB + sid
    D = zbuf_v.shape[1]

    for r in range(16):
        for c in range(D // 16):
            zbuf_v[r, pl.ds(c * 16, 16)] = jnp.zeros((16,), jnp.float32)
    zd = []
    for t in range(ROWS_PER_SUB // 16):
        zd.append(pltpu.async_copy(
            zbuf_v, acc_sh.at[pl.ds(sid * ROWS_PER_SUB + t * 16, 16)], sem_g))
    for d in zd:
        d.wait()

    pltpu.sync_copy(isrc_hbm.at[w], isrc_v)
    pltpu.sync_copy(idst_hbm.at[w], idst_v)
    plsc.subcore_barrier()

    pltpu.async_copy(y_hbm.at[isrc_v.at[0]], rows_v.at[0], sem_g)

    def chunk(j, b):
        @pl.when(j + 1 < CH)
        def _():
            pltpu.async_copy(y_hbm.at[isrc_v.at[j + 1]], rows_v.at[1 - b], sem_g)
        pltpu.make_async_copy(y_hbm.at[isrc_v.at[j]], rows_v.at[b], sem_g).wait()
        pltpu.sync_copy(rows_v.at[b], acc_sh.at[idst_v.at[j]], add=True)

    def body(it, _):
        chunk(2 * it, 0)
        chunk(2 * it + 1, 1)
        return 0

    lax.fori_loop(0, CH // 2, body, 0)
    plsc.subcore_barrier()

    pltpu.sync_copy(acc_sh.at[pl.ds(sid * ROWS_PER_SUB, ROWS_PER_SUB)],
                    out_hbm.at[cid].at[pl.ds(sid * ROWS_PER_SUB, ROWS_PER_SUB)])


def _sc_agg(y, isrc, idst, D):
    mesh = plsc.VectorSubcoreMesh(core_axis_name="c", subcore_axis_name="s")
    return pl.kernel(
        _agg_body,
        out_type=jax.ShapeDtypeStruct((2, NPAD, D), jnp.float32),
        mesh=mesh,
        scratch_types=[
            pltpu.VMEM((CH, K), jnp.int32),
            pltpu.VMEM((CH, K), jnp.int32),
            pltpu.VMEM((2, K, D), jnp.float32),
            pltpu.VMEM((16, D), jnp.float32),
            pltpu.VMEM_SHARED((NPAD, D), jnp.float32),
            pltpu.SemaphoreType.DMA,
        ],
    )(y, isrc, idst)


def _deg_body(idst_hbm, out_hbm, idst_v, ones_v, zbuf_v, acc_sh, sem_g):
    cid = lax.axis_index("c")
    sid = lax.axis_index("s")
    w = cid * NSUB + sid

    for c in range(8):
        zbuf_v[pl.ds(c * 16, 16)] = jnp.zeros((16,), jnp.float32)
        ones_v[pl.ds(c * 16, 16)] = jnp.ones((16,), jnp.float32)
    zd = []
    for t in range(ROWS_PER_SUB // 128):
        zd.append(pltpu.async_copy(
            zbuf_v, acc_sh.at[pl.ds(sid * ROWS_PER_SUB + t * 128, 128)], sem_g))
    for d in zd:
        d.wait()

    pltpu.sync_copy(idst_hbm.at[w], idst_v)
    plsc.subcore_barrier()

    def body(j, _):
        pltpu.sync_copy(ones_v, acc_sh.at[idst_v.at[j]], add=True)
        return 0

    lax.fori_loop(0, CH, body, 0)
    plsc.subcore_barrier()

    pltpu.sync_copy(acc_sh.at[pl.ds(sid * ROWS_PER_SUB, ROWS_PER_SUB)],
                    out_hbm.at[cid].at[pl.ds(sid * ROWS_PER_SUB, ROWS_PER_SUB)])


def _sc_deg(idst):
    mesh = plsc.VectorSubcoreMesh(core_axis_name="c", subcore_axis_name="s")
    return pl.kernel(
        _deg_body,
        out_type=jax.ShapeDtypeStruct((2, NPAD), jnp.float32),
        mesh=mesh,
        scratch_types=[
            pltpu.VMEM((CH, K), jnp.int32),
            pltpu.VMEM((K,), jnp.float32),
            pltpu.VMEM((128,), jnp.float32),
            pltpu.VMEM_SHARED((NPAD,), jnp.float32),
            pltpu.SemaphoreType.DMA,
        ],
    )(idst)


def _adj_body(l_ref, lt_ref, out_ref):
    acc = lax.dot_general(l_ref[...], lt_ref[...], (((1,), (0,)), ((), ())),
                          preferred_element_type=jnp.float32)
    out_ref[...] = jax.nn.sigmoid(acc)


def _adj_recon(el):
    elt = el.T
    n = el.shape[0]
    grid = (pl.cdiv(n, BM), pl.cdiv(n, BN))
    return pl.pallas_call(
        _adj_body,
        grid=grid,
        in_specs=[
            pl.BlockSpec((BM, el.shape[1]), lambda i, j: (i, 0)),
            pl.BlockSpec((el.shape[1], BN), lambda i, j: (0, j)),
        ],
        out_specs=pl.BlockSpec((BM, BN), lambda i, j: (i, j)),
        out_shape=jax.ShapeDtypeStruct((n, n), jnp.float32),
    )(el, elt)


def kernel(x, edge_index, batch, W_g1, b_g1, W_g2, b_g2, W_d1, b_d1, W_d2, b_d2, W_e, b_e):
    src = edge_index[0].astype(jnp.int32)
    dst = edge_index[1].astype(jnp.int32)
    isrc = _pad_idx(src, 0)
    idst = _pad_idx(dst, TRASH)

    degp = _sc_deg(idst)
    deg = degp[0, :N] + degp[1, :N] + 1.0
    dinv = deg ** -0.5

    y1 = (x @ W_g1) * dinv[:, None]
    aggp1 = _sc_agg(y1, isrc, idst, 128)
    z1 = jax.nn.relu((aggp1[0, :N] + aggp1[1, :N] + y1) * dinv[:, None] + b_g1)

    y2 = (z1 @ W_g2) * dinv[:, None]
    y2p = jnp.concatenate([y2, jnp.zeros_like(y2)], axis=1)
    aggp2 = _sc_agg(y2p, isrc, idst, 128)
    z = jax.nn.relu((aggp2[0, :N, :64] + aggp2[1, :N, :64] + y2) * dinv[:, None] + b_g2)

    h = jax.nn.relu(z @ W_d1 + b_d1)
    x_recon = h @ W_d2 + b_d2
    edge_logits = z @ W_e + b_e
    adj_recon = _adj_recon(edge_logits)
    z_g = jax.ops.segment_max(z, batch, num_segments=64)
    return (z, z_g, x_recon, adj_recon)
